# baseline (speedup 1.0000x reference)
import functools
import math

import jax
import jax.numpy as jnp
from jax import lax
from jax.experimental import pallas as pl
from jax.experimental.pallas import tpu as pltpu
from jax.experimental.pallas import tpu_sc as plsc

_NC = 2
_NS = 16
_LANES = 16

_STREAM = 128


@functools.lru_cache(maxsize=None)
def _build(BF, V, D, F):
    NW = _NC * _NS
    per_w = BF // NW
    pat = (F * _LANES) // math.gcd(F, _LANES)
    rows_per_pat = pat // F
    NIDX = 1664
    assert NIDX % pat == 0 and NIDX % _STREAM == 0 and per_w % NIDX == 0
    NST = NIDX // _STREAM
    G = per_w // NIDX

    mesh = plsc.VectorSubcoreMesh(core_axis_name="c", subcore_axis_name="s")

    @functools.partial(
        pl.kernel,
        mesh=mesh,
        compiler_params=pltpu.CompilerParams(use_tc_tiling_on_sc=False),
        out_type=jax.ShapeDtypeStruct((BF, D), jnp.float32),
        scratch_types=[
            pltpu.VMEM((NIDX,), jnp.int32),
            pltpu.VMEM((pat,), jnp.int32),
            pltpu.VMEM((NIDX,), jnp.int32),
            pltpu.VMEM((NIDX, D), jnp.float32),
            pltpu.SemaphoreType.DMA,
        ],
    )
    def k(xf, table, offp, out, x_v, off_v, idx_v, rows_v, sem):
        wid = lax.axis_index("s") * _NC + lax.axis_index("c")
        wbase = wid * per_w
        pltpu.sync_copy(offp, off_v)

        def body(g, carry):
            base = wbase + g * NIDX
            pltpu.sync_copy(xf.at[pl.ds(base, NIDX)], x_v)
            for s in range(NIDX // pat):
                for j in range(pat // _LANES):
                    sl = pl.ds(s * pat + j * _LANES, _LANES)
                    idx_v[sl] = x_v[sl] + off_v[pl.ds(j * _LANES, _LANES)]
            cps = [
                pltpu.async_copy(
                    table.at[idx_v.at[pl.ds(j * _STREAM, _STREAM)]],
                    rows_v.at[pl.ds(j * _STREAM, _STREAM)],
                    sem,
                )
                for j in range(NST)
            ]
            for cp in cps:
                cp.wait()
            pltpu.sync_copy(rows_v, out.at[pl.ds(base, NIDX)])
            return carry

        lax.fori_loop(0, G, body, 0)

    return k


def kernel(x, table, offsets):
    B, F = x.shape
    V, D = table.shape
    xf = x.reshape(B * F).astype(jnp.int32)
    pat_reps = _LANES // math.gcd(F, _LANES)
    offp = jnp.tile(offsets.astype(jnp.int32), pat_reps)
    out = _build(B * F, V, D, F)(xf, table, offp)
    return out.reshape(B, F, D)


# speedup vs baseline: 1.0111x; 1.0111x over previous
import functools
import math

import jax
import jax.numpy as jnp
from jax import lax
from jax.experimental import pallas as pl
from jax.experimental.pallas import tpu as pltpu
from jax.experimental.pallas import tpu_sc as plsc

_NC = 2
_NS = 16
_LANES = 16

_STREAM = 128


@functools.lru_cache(maxsize=None)
def _build(BF, V, D, F):
    NW = _NC * _NS
    per_w = BF // NW
    pat = (F * _LANES) // math.gcd(F, _LANES)
    NIDX = 1664
    assert NIDX % pat == 0 and NIDX % _STREAM == 0 and per_w % NIDX == 0
    NST = NIDX // _STREAM
    G = per_w // NIDX
    SUBS = NIDX // pat

    mesh = plsc.VectorSubcoreMesh(core_axis_name="c", subcore_axis_name="s")

    @functools.partial(
        pl.kernel,
        mesh=mesh,
        compiler_params=pltpu.CompilerParams(use_tc_tiling_on_sc=False),
        out_type=jax.ShapeDtypeStruct((BF, D), jnp.float32),
        scratch_types=[
            [pltpu.VMEM((NIDX,), jnp.int32)] * 2,
            pltpu.VMEM((pat,), jnp.int32),
            [pltpu.VMEM((NIDX,), jnp.int32)] * 2,
            [pltpu.VMEM((NIDX, D), jnp.float32)] * 2,
            [pltpu.SemaphoreType.DMA] * 2,
            [pltpu.SemaphoreType.DMA] * 2,
            [pltpu.SemaphoreType.DMA] * 2,
        ],
    )
    def k(xf, table, offp, out, x_v, off_v, idx_v, rows_v, sx, sg, so):
        wid = lax.axis_index("s") * _NC + lax.axis_index("c")
        wbase = wid * per_w
        pltpu.sync_copy(offp, off_v)

        def compute_idx(p):
            def sub_body(s, carry):
                o0 = s * pat
                for j in range(pat // _LANES):
                    sl = pl.ds(o0 + j * _LANES, _LANES)
                    idx_v[p][sl] = (
                        x_v[p][sl] + off_v[pl.ds(j * _LANES, _LANES)]
                    )
                return carry

            lax.fori_loop(0, SUBS, sub_body, 0)

        def x_copy(g, p):
            return pltpu.async_copy(
                xf.at[pl.ds(wbase + g * NIDX, NIDX)], x_v[p], sx[p]
            )

        def gather(g, p):
            return [
                pltpu.async_copy(
                    table.at[idx_v[p].at[pl.ds(j * _STREAM, _STREAM)]],
                    rows_v[p].at[pl.ds(j * _STREAM, _STREAM)],
                    sg[p],
                )
                for j in range(NST)
            ]

        def writeback(g, p):
            return pltpu.async_copy(
                rows_v[p], out.at[pl.ds(wbase + g * NIDX, NIDX)], so[p]
            )

        xcps, gcps, wcps = {}, {}, {}

        x_copy(0, 0).wait()
        compute_idx(0)
        if G > 1:
            xcps[1] = x_copy(1, 1)

        for g in range(G):
            p, q = g % 2, 1 - (g % 2)
            if g >= 2:
                wcps[g - 2].wait()
            gcps[g] = gather(g, p)
            if g + 2 < G:
                xcps[g + 2] = x_copy(g + 2, p)
            if g >= 1:
                for cp in gcps[g - 1]:
                    cp.wait()
                wcps[g - 1] = writeback(g - 1, q)
            if g + 1 < G:
                xcps[g + 1].wait()
                compute_idx(q)

        for cp in gcps[G - 1]:
            cp.wait()
        wcps[G - 1] = writeback(G - 1, (G - 1) % 2)
        wcps[G - 2].wait()
        wcps[G - 1].wait()

    return k


def kernel(x, table, offsets):
    B, F = x.shape
    V, D = table.shape
    xf = x.reshape(B * F).astype(jnp.int32)
    pat_reps = _LANES // math.gcd(F, _LANES)
    offp = jnp.tile(offsets.astype(jnp.int32), pat_reps)
    out = _build(B * F, V, D, F)(xf, table, offp)
    return out.reshape(B, F, D)
